# baseline (speedup 1.0000x reference)
import functools

import jax
import jax.numpy as jnp
from jax import lax
from jax.experimental import pallas as pl
from jax.experimental.pallas import tpu as pltpu
from jax.experimental.pallas import tpu_sc as plsc

N = 100000
E = 1600000
G = 128

NP = 100352
ER = 12800
EP = ER * 128
TRASH = 100000

_F32 = jnp.float32
_MESH = plsc.VectorSubcoreMesh(core_axis_name="c", subcore_axis_name="s")
_SC_PARAMS = pltpu.CompilerParams(use_tc_tiling_on_sc=False)


def _sc_degree(dst2d):
    CL_PT = NP // 16
    ZD = 784
    NCPD = CL_PT // ZD
    TILE_ROWS = ER // 32
    W_ROWS = 16
    N_IT = TILE_ROWS // W_ROWS

    @functools.partial(
        pl.kernel,
        out_type=jax.ShapeDtypeStruct((2, NP), _F32),
        mesh=_MESH,
        compiler_params=_SC_PARAMS,
        scratch_types=[
            pltpu.VMEM((W_ROWS, 128), jnp.int32),
            pltpu.VMEM((128,), _F32),
            pltpu.VMEM((ZD,), _F32),
            pltpu.VMEM_SHARED((NP,), _F32),
            pltpu.SemaphoreType.DMA,
        ],
    )
    def deg_kernel(dst_hbm, out_hbm, dstv, ones, stg, acc, ssem):
        c = lax.axis_index("c")
        s = lax.axis_index("s")

        @pl.loop(0, 128, step=16)
        def _(k):
            ones[pl.ds(k, 16)] = jnp.full((16,), 1.0, _F32)

        @pl.loop(0, ZD, step=16)
        def _(k):
            stg[pl.ds(k, 16)] = jnp.zeros((16,), _F32)

        for q in range(NCPD):
            pltpu.sync_copy(stg, acc.at[pl.ds(s * CL_PT + q * ZD, ZD)])
        plsc.subcore_barrier()

        @pl.loop(0, N_IT)
        def _(it):
            r0 = c * (ER // 2) + s * TILE_ROWS + it * W_ROWS
            pltpu.sync_copy(dst_hbm.at[pl.ds(r0, W_ROWS)], dstv)
            copies = [
                pltpu.async_copy(ones, acc.at[dstv.at[j]], ssem, add=True)
                for j in range(W_ROWS)
            ]
            for cp in copies:
                cp.wait()

        plsc.subcore_barrier()
        for q in range(NCPD):
            pltpu.sync_copy(acc.at[pl.ds(s * CL_PT + q * ZD, ZD)], stg)
            pltpu.sync_copy(
                stg, out_hbm.at[c, pl.ds(s * CL_PT + q * ZD, ZD)])

    return deg_kernel(dst2d)


def _sc_segsum_slices(u_slices, src2d, dst2d):
    K = len(u_slices)
    SF = 16
    W = 4
    WE = W * 128
    GW = 10
    GR = GW * W
    TILE_ROWS = ER // 32
    NGRP = TILE_ROWS // GR
    CL_PT = NP // 16
    NCP = CL_PT // WE
    CL_TAIL = CL_PT - NCP * WE

    @functools.partial(
        pl.kernel,
        out_type=[jax.ShapeDtypeStruct((2, NP, SF), _F32) for _ in range(K)],
        mesh=_MESH,
        compiler_params=_SC_PARAMS,
        scratch_types=[
            pltpu.VMEM((GR, 128), jnp.int32),
            pltpu.VMEM((GR, 128), jnp.int32),
            pltpu.VMEM((2, WE, SF), _F32),
            pltpu.VMEM_SHARED((NP, SF), _F32),
            pltpu.SemaphoreType.DMA,
            pltpu.SemaphoreType.DMA,
            pltpu.SemaphoreType.DMA,
            pltpu.SemaphoreType.DMA,
        ],
    )
    def seg_kernel(*refs):
        us = refs[:K]
        src_hbm, dst_hbm = refs[K], refs[K + 1]
        outs = refs[K + 2:2 * K + 2]
        sidx, dstv, rows, acc, g0, g1, s0, s1 = refs[2 * K + 2:]
        gsems = (g0, g1)
        ssems = (s0, s1)
        c = lax.axis_index("c")
        s = lax.axis_index("s")
        row0 = c * (ER // 2) + s * TILE_ROWS

        for sl in range(K):
            u_hbm = us[sl]
            out_hbm = outs[sl]

            @pl.loop(0, WE)
            def _(r):
                rows[0, r, pl.ds(0, 16)] = jnp.zeros((16,), _F32)

            for q in range(NCP):
                pltpu.sync_copy(rows.at[0],
                                acc.at[pl.ds(s * CL_PT + q * WE, WE)])
            pltpu.sync_copy(
                rows.at[0, pl.ds(0, CL_TAIL)],
                acc.at[pl.ds(s * CL_PT + NCP * WE, CL_TAIL)])
            plsc.subcore_barrier()

            def fire_g(k, rb, u_hbm=u_hbm):
                for j in range(W):
                    pltpu.async_copy(u_hbm.at[sidx.at[k * W + j]],
                                     rows.at[rb, pl.ds(j * 128, 128)],
                                     gsems[rb])

            def wait_g(k, rb, u_hbm=u_hbm):
                for j in range(W):
                    pltpu.make_async_copy(
                        u_hbm.at[sidx.at[k * W + j]],
                        rows.at[rb, pl.ds(j * 128, 128)], gsems[rb]).wait()

            def fire_s(k, rb):
                for j in range(W):
                    pltpu.async_copy(rows.at[rb, pl.ds(j * 128, 128)],
                                     acc.at[dstv.at[k * W + j]], ssems[rb],
                                     add=True)

            def wait_s(k, rb):
                for j in range(W):
                    pltpu.make_async_copy(
                        rows.at[rb, pl.ds(j * 128, 128)],
                        acc.at[dstv.at[k * W + j]], ssems[rb]).wait()

            @pl.loop(0, NGRP)
            def _(gi):
                r0 = row0 + gi * GR
                pltpu.sync_copy(src_hbm.at[pl.ds(r0, GR)], sidx)
                pltpu.sync_copy(dst_hbm.at[pl.ds(r0, GR)], dstv)
                fire_g(0, 0)
                for k in range(GW):
                    rb = k % 2
                    if k >= 1:
                        wait_s(k - 1, rb ^ 1)
                    if k + 1 < GW:
                        fire_g(k + 1, rb ^ 1)
                    wait_g(k, rb)
                    fire_s(k, rb)
                wait_s(GW - 1, (GW - 1) % 2)

            plsc.subcore_barrier()
            for q in range(NCP):
                pltpu.sync_copy(acc.at[pl.ds(s * CL_PT + q * WE, WE)],
                                rows.at[1])
                pltpu.sync_copy(
                    rows.at[1], out_hbm.at[c, pl.ds(s * CL_PT + q * WE, WE)])
            pltpu.sync_copy(
                acc.at[pl.ds(s * CL_PT + NCP * WE, CL_TAIL)],
                rows.at[1, pl.ds(0, CL_TAIL)])
            pltpu.sync_copy(
                rows.at[1, pl.ds(0, CL_TAIL)],
                out_hbm.at[c, pl.ds(s * CL_PT + NCP * WE, CL_TAIL)])
            plsc.subcore_barrier()

    res = seg_kernel(*u_slices, src2d, dst2d)
    return list(res) if isinstance(res, (tuple, list)) else [res]


def _tc_first(x, W1, deg):
    BR = 2000

    def body(x_ref, w_ref, deg_ref, u_ref, dinv_ref):
        dg = deg_ref[0, :, :] + deg_ref[1, :, :]
        dinv = lax.rsqrt(dg + 1.0)
        xx = x_ref[...]
        w = w_ref[...]
        h = (xx[:, 0:1] * w[0:1, :] + xx[:, 1:2] * w[1:2, :]
             + xx[:, 2:3] * w[2:3, :])
        u_ref[...] = dinv * h
        dinv_ref[...] = dinv

    return pl.pallas_call(
        body,
        grid=(N // BR,),
        in_specs=[
            pl.BlockSpec((BR, 3), lambda i: (i, 0)),
            pl.BlockSpec((3, 16), lambda i: (0, 0)),
            pl.BlockSpec((2, BR, 1), lambda i: (0, i, 0)),
        ],
        out_specs=[
            pl.BlockSpec((BR, 16), lambda i: (i, 0)),
            pl.BlockSpec((BR, 1), lambda i: (i, 0)),
        ],
        out_shape=[
            jax.ShapeDtypeStruct((N, 16), _F32),
            jax.ShapeDtypeStruct((N, 1), _F32),
        ],
    )(x, W1, deg)


def _tc_stats(S_slices, u_slices, dinv, b2d, F):
    K = F // 16
    BR = 2000
    NB = N // BR

    def body(*refs):
        s_refs = refs[:K]
        u_refs = refs[K:2 * K]
        dinv_ref, b_ref, out_ref, acc = refs[2 * K:]
        i = pl.program_id(0)

        @pl.when(i == 0)
        def _():
            acc[...] = jnp.zeros_like(acc)

        dinv = dinv_ref[...]
        for sl in range(K):
            svals = s_refs[sl][0, :, :] + s_refs[sl][1, :, :]
            t = dinv * (svals + u_refs[sl][...]) + b_ref[:, sl * 16:sl * 16 + 16]
            acc[0:1, sl * 16:sl * 16 + 16] += jnp.sum(t, axis=0, keepdims=True)
            acc[1:2, sl * 16:sl * 16 + 16] += jnp.sum(t * t, axis=0,
                                                      keepdims=True)

        @pl.when(i == NB - 1)
        def _():
            out_ref[...] = acc[...]

    return pl.pallas_call(
        body,
        grid=(NB,),
        in_specs=(
            [pl.BlockSpec((2, BR, 16), lambda i: (0, i, 0)) for _ in range(K)]
            + [pl.BlockSpec((BR, 16), lambda i: (i, 0)) for _ in range(K)]
            + [pl.BlockSpec((BR, 1), lambda i: (i, 0)),
               pl.BlockSpec((1, F), lambda i: (0, 0))]
        ),
        out_specs=pl.BlockSpec((8, F), lambda i: (0, 0)),
        out_shape=jax.ShapeDtypeStruct((8, F), _F32),
        scratch_shapes=[pltpu.VMEM((8, F), _F32)],
    )(*S_slices, *u_slices, dinv, b2d)


def _tc_apply(S_slices, u_slices, dinv, b2d, g2d, be2d, stats, Wn, F, F2):
    K = F // 16
    K2 = F2 // 16
    BR = 2000

    def body(*refs):
        s_refs = refs[:K]
        u_refs = refs[K:2 * K]
        dinv_ref, b_ref, g_ref, be_ref, st_ref, w_ref = refs[2 * K:2 * K + 6]
        out_refs = refs[2 * K + 6:]
        dinv = dinv_ref[...]
        mu = st_ref[0:1, :] * (1.0 / N)
        var = st_ref[1:2, :] * (1.0 / N) - mu * mu
        a = g_ref[...] * lax.rsqrt(var + 1e-5)
        hs = []
        for sl in range(K):
            svals = s_refs[sl][0, :, :] + s_refs[sl][1, :, :]
            t = dinv * (svals + u_refs[sl][...]) + b_ref[:, sl * 16:sl * 16 + 16]
            hs.append(jnp.maximum(
                a[:, sl * 16:sl * 16 + 16] * (t - mu[:, sl * 16:sl * 16 + 16])
                + be_ref[:, sl * 16:sl * 16 + 16], 0.0))
        h = jnp.concatenate(hs, axis=1)
        un = dinv * jnp.dot(h, w_ref[...], preferred_element_type=_F32)
        for sl in range(K2):
            out_refs[sl][...] = un[:, sl * 16:sl * 16 + 16]

    return pl.pallas_call(
        body,
        grid=(N // BR,),
        in_specs=(
            [pl.BlockSpec((2, BR, 16), lambda i: (0, i, 0)) for _ in range(K)]
            + [pl.BlockSpec((BR, 16), lambda i: (i, 0)) for _ in range(K)]
            + [pl.BlockSpec((BR, 1), lambda i: (i, 0)),
               pl.BlockSpec((1, F), lambda i: (0, 0)),
               pl.BlockSpec((1, F), lambda i: (0, 0)),
               pl.BlockSpec((1, F), lambda i: (0, 0)),
               pl.BlockSpec((8, F), lambda i: (0, 0)),
               pl.BlockSpec((F, F2), lambda i: (0, 0))]
        ),
        out_specs=[pl.BlockSpec((BR, 16), lambda i: (i, 0))
                   for _ in range(K2)],
        out_shape=[jax.ShapeDtypeStruct((N, 16), _F32) for _ in range(K2)],
    )(*S_slices, *u_slices, dinv, b2d, g2d, be2d, stats, Wn)


def _tc_hist(batch2d):
    BR = 2000
    NB = N // BR

    def body(bt_ref, out_ref, acc):
        i = pl.program_id(0)

        @pl.when(i == 0)
        def _():
            acc[...] = jnp.zeros_like(acc)

        ids = bt_ref[...]
        gidx = lax.broadcasted_iota(jnp.int32, (BR, G), 1)
        acc[0:1, :] += jnp.sum((ids == gidx).astype(jnp.int32), axis=0,
                               keepdims=True)

        @pl.when(i == NB - 1)
        def _():
            out_ref[...] = acc[0:1, :]

    return pl.pallas_call(
        body,
        grid=(NB,),
        in_specs=[pl.BlockSpec((BR, 1), lambda i: (i, 0))],
        out_specs=pl.BlockSpec((1, G), lambda i: (0, 0)),
        out_shape=jax.ShapeDtypeStruct((1, G), jnp.int32),
        scratch_shapes=[pltpu.VMEM((8, G), jnp.int32)],
    )(batch2d)


def _tc_pool_head(S_slices, u_slices, dinv, b2d, g2d, be2d, stats, hist,
                  batch2d, Wf, bf2d):
    K = 4
    BR = 2000
    NB = N // BR
    F = 64
    CH = 64

    def body(*refs):
        s_refs = refs[:K]
        u_refs = refs[K:2 * K]
        (dinv_ref, b_ref, g_ref, be_ref, st_ref, ht_ref, bt_ref, wf_ref,
         bf_ref, lo_ref, pr_ref, hblk, starts, pacc) = refs[2 * K:]
        i = pl.program_id(0)

        @pl.when(i == 0)
        def _():
            sacc = 0
            for g in range(G):
                starts[g] = sacc
                sacc = sacc + ht_ref[0, g]
            starts[G] = sacc
            pacc[...] = jnp.full((G, F), -jnp.inf, _F32)

        dinv = dinv_ref[...]
        mu = st_ref[0:1, :] * (1.0 / N)
        var = st_ref[1:2, :] * (1.0 / N) - mu * mu
        a = g_ref[...] * lax.rsqrt(var + 1e-5)
        for sl in range(K):
            svals = s_refs[sl][0, :, :] + s_refs[sl][1, :, :]
            t = (dinv * (svals + u_refs[sl][...])
                 + b_ref[:, sl * 16:sl * 16 + 16])
            h = jnp.maximum(
                a[:, sl * 16:sl * 16 + 16]
                * (t - mu[:, sl * 16:sl * 16 + 16])
                + be_ref[:, sl * 16:sl * 16 + 16], 0.0)
            hblk[pl.ds(0, BR), sl * 16:sl * 16 + 16] = h

        blk_lo = i * BR
        gfirst = bt_ref[0, 0]
        glast = bt_ref[BR - 1, 0]

        def graph_body(g, carry):
            s_g = jnp.maximum(starts[g], blk_lo) - blk_lo
            e_g = jnp.minimum(starts[g + 1], blk_lo + BR) - blk_lo
            nch = (e_g - s_g + CH - 1) // CH

            def chunk_body(k, mx):
                j = s_g + k * CH
                hv = hblk[pl.ds(j, CH), :]
                rid = j + lax.broadcasted_iota(jnp.int32, (CH, F), 0)
                return jnp.maximum(mx, jnp.where(rid < e_g, hv, -jnp.inf))

            acc0 = jnp.full((CH, F), -jnp.inf, _F32)
            mx = lax.fori_loop(0, nch, chunk_body, acc0)
            cur = pacc[pl.ds(g, 1), :]
            pacc[pl.ds(g, 1), :] = jnp.maximum(
                cur, jnp.max(mx, axis=0, keepdims=True))
            return carry

        lax.fori_loop(gfirst, glast + 1, graph_body, 0)

        @pl.when(i == NB - 1)
        def _():
            p = pacc[...]
            logits = jnp.dot(p, wf_ref[...],
                             preferred_element_type=_F32) + bf_ref[...]
            lo_ref[...] = logits
            mxl = jnp.max(logits, axis=1, keepdims=True)
            e = jnp.exp(logits - mxl)
            pr_ref[...] = e / jnp.sum(e, axis=1, keepdims=True)

    return pl.pallas_call(
        body,
        grid=(NB,),
        in_specs=(
            [pl.BlockSpec((2, BR, 16), lambda i: (0, i, 0))
             for _ in range(K)]
            + [pl.BlockSpec((BR, 16), lambda i: (i, 0)) for _ in range(K)]
            + [pl.BlockSpec((BR, 1), lambda i: (i, 0)),
               pl.BlockSpec((1, F), lambda i: (0, 0)),
               pl.BlockSpec((1, F), lambda i: (0, 0)),
               pl.BlockSpec((1, F), lambda i: (0, 0)),
               pl.BlockSpec((8, F), lambda i: (0, 0)),
               pl.BlockSpec(memory_space=pltpu.SMEM),
               pl.BlockSpec((BR, 1), lambda i: (i, 0)),
               pl.BlockSpec((F, 10), lambda i: (0, 0)),
               pl.BlockSpec((1, 10), lambda i: (0, 0))]
        ),
        out_specs=[
            pl.BlockSpec((G, 10), lambda i: (0, 0)),
            pl.BlockSpec((G, 10), lambda i: (0, 0)),
        ],
        out_shape=[
            jax.ShapeDtypeStruct((G, 10), _F32),
            jax.ShapeDtypeStruct((G, 10), _F32),
        ],
        scratch_shapes=[
            pltpu.VMEM((BR + CH, F), _F32),
            pltpu.SMEM((G + 8,), jnp.int32),
            pltpu.VMEM((G, F), _F32),
        ],
    )(*S_slices, *u_slices, dinv, b2d, g2d, be2d, stats, hist, batch2d,
      Wf, bf2d)


def kernel(x, edge_index, batch, W1, b1, g1, be1, W2, b2, g2, be2,
           W3, b3, g3, be3, Wf, bf):
    src = edge_index[0].astype(jnp.int32)
    dst = edge_index[1].astype(jnp.int32)
    pad = EP - E
    src2d = jnp.concatenate(
        [src, jnp.zeros((pad,), jnp.int32)]).reshape(ER, 128)
    dst2d = jnp.concatenate(
        [dst, jnp.full((pad,), TRASH, jnp.int32)]).reshape(ER, 128)

    deg = _sc_degree(dst2d)
    deg3 = deg[:, :N, None]

    u1, dinv = _tc_first(x, W1, deg3)

    b1r, g1r, be1r = b1[None, :], g1[None, :], be1[None, :]
    b2r, g2r, be2r = b2[None, :], g2[None, :], be2[None, :]
    b3r, g3r, be3r = b3[None, :], g3[None, :], be3[None, :]
    bfr = bf[None, :]
    batch2d = batch.astype(jnp.int32)[:, None]

    S1 = _sc_segsum_slices([u1], src2d, dst2d)
    st1 = _tc_stats(S1, [u1], dinv, b1r, 16)
    u2s = _tc_apply(S1, [u1], dinv, b1r, g1r, be1r, st1, W2, 16, 32)

    S2 = _sc_segsum_slices(list(u2s), src2d, dst2d)
    st2 = _tc_stats(S2, list(u2s), dinv, b2r, 32)
    u3s = _tc_apply(S2, list(u2s), dinv, b2r, g2r, be2r, st2, W3, 32, 64)

    S3 = _sc_segsum_slices(list(u3s), src2d, dst2d)
    st3 = _tc_stats(S3, list(u3s), dinv, b3r, 64)
    hist = _tc_hist(batch2d)
    logits, probs = _tc_pool_head(S3, list(u3s), dinv, b3r, g3r, be3r, st3,
                                  hist, batch2d, Wf, bfr)
    return (logits, probs)


# speedup vs baseline: 1.4669x; 1.4669x over previous
import functools

import jax
import jax.numpy as jnp
from jax import lax
from jax.experimental import pallas as pl
from jax.experimental.pallas import tpu as pltpu
from jax.experimental.pallas import tpu_sc as plsc

N = 100000
E = 1600000
G = 128

NP = 100352
ER = 12800
EP = ER * 128
TRASH = 100000

_F32 = jnp.float32
_MESH = plsc.VectorSubcoreMesh(core_axis_name="c", subcore_axis_name="s")
_SC_PARAMS = pltpu.CompilerParams(use_tc_tiling_on_sc=False)


def _sc_degree(dst2d):
    CL_PT = NP // 16
    ZD = 784
    NCPD = CL_PT // ZD
    TILE_ROWS = ER // 32
    W_ROWS = 16
    N_IT = TILE_ROWS // W_ROWS

    @functools.partial(
        pl.kernel,
        out_type=jax.ShapeDtypeStruct((2, NP), _F32),
        mesh=_MESH,
        compiler_params=_SC_PARAMS,
        scratch_types=[
            pltpu.VMEM((W_ROWS, 128), jnp.int32),
            pltpu.VMEM((128,), _F32),
            pltpu.VMEM((ZD,), _F32),
            pltpu.VMEM_SHARED((NP,), _F32),
            pltpu.SemaphoreType.DMA,
        ],
    )
    def deg_kernel(dst_hbm, out_hbm, dstv, ones, stg, acc, ssem):
        c = lax.axis_index("c")
        s = lax.axis_index("s")

        @pl.loop(0, 128, step=16)
        def _(k):
            ones[pl.ds(k, 16)] = jnp.full((16,), 1.0, _F32)

        @pl.loop(0, ZD, step=16)
        def _(k):
            stg[pl.ds(k, 16)] = jnp.zeros((16,), _F32)

        for q in range(NCPD):
            pltpu.sync_copy(stg, acc.at[pl.ds(s * CL_PT + q * ZD, ZD)])
        plsc.subcore_barrier()

        @pl.loop(0, N_IT)
        def _(it):
            r0 = c * (ER // 2) + s * TILE_ROWS + it * W_ROWS
            pltpu.sync_copy(dst_hbm.at[pl.ds(r0, W_ROWS)], dstv)
            copies = [
                pltpu.async_copy(ones, acc.at[dstv.at[j]], ssem, add=True)
                for j in range(W_ROWS)
            ]
            for cp in copies:
                cp.wait()

        plsc.subcore_barrier()
        for q in range(NCPD):
            pltpu.sync_copy(acc.at[pl.ds(s * CL_PT + q * ZD, ZD)], stg)
            pltpu.sync_copy(
                stg, out_hbm.at[c, pl.ds(s * CL_PT + q * ZD, ZD)])

    return deg_kernel(dst2d)


def _sc_segsum_slices(u_slices, src2d, dst2d):
    K = len(u_slices)
    SF = 16
    W = 4
    WE = W * 128
    GW = 10
    GR = GW * W
    R0 = 280
    R1 = ER // 16 - R0
    CL_PT = NP // 16
    NCP = CL_PT // WE
    CL_TAIL = CL_PT - NCP * WE

    @functools.partial(
        pl.kernel,
        out_type=[jax.ShapeDtypeStruct((2, NP, SF), _F32) for _ in range(K)],
        mesh=_MESH,
        compiler_params=_SC_PARAMS,
        scratch_types=[
            pltpu.VMEM((GR, 128), jnp.int32),
            pltpu.VMEM((GR, 128), jnp.int32),
            pltpu.VMEM((2, WE, SF), _F32),
            pltpu.VMEM_SHARED((NP, SF), _F32),
            pltpu.SemaphoreType.DMA,
            pltpu.SemaphoreType.DMA,
            pltpu.SemaphoreType.DMA,
            pltpu.SemaphoreType.DMA,
        ],
    )
    def seg_kernel(*refs):
        us = refs[:K]
        src_hbm, dst_hbm = refs[K], refs[K + 1]
        outs = refs[K + 2:2 * K + 2]
        sidx, dstv, rows, acc, g0, g1, s0, s1 = refs[2 * K + 2:]
        gsems = (g0, g1)
        ssems = (s0, s1)
        c = lax.axis_index("c")
        s = lax.axis_index("s")
        row0 = jnp.where(c == 0, s * R0, 16 * R0 + s * R1)
        ngrp = jnp.where(c == 0, R0 // GR, R1 // GR)

        for sl in range(K):
            u_hbm = us[sl]
            out_hbm = outs[sl]

            @pl.loop(0, WE)
            def _(r):
                rows[0, r, pl.ds(0, 16)] = jnp.zeros((16,), _F32)

            for q in range(NCP):
                pltpu.sync_copy(rows.at[0],
                                acc.at[pl.ds(s * CL_PT + q * WE, WE)])
            pltpu.sync_copy(
                rows.at[0, pl.ds(0, CL_TAIL)],
                acc.at[pl.ds(s * CL_PT + NCP * WE, CL_TAIL)])
            plsc.subcore_barrier()

            def fire_g(k, rb, u_hbm=u_hbm):
                for j in range(W):
                    pltpu.async_copy(u_hbm.at[sidx.at[k * W + j]],
                                     rows.at[rb, pl.ds(j * 128, 128)],
                                     gsems[rb])

            def wait_g(k, rb, u_hbm=u_hbm):
                for j in range(W):
                    pltpu.make_async_copy(
                        u_hbm.at[sidx.at[k * W + j]],
                        rows.at[rb, pl.ds(j * 128, 128)], gsems[rb]).wait()

            def fire_s(k, rb):
                for j in range(W):
                    pltpu.async_copy(rows.at[rb, pl.ds(j * 128, 128)],
                                     acc.at[dstv.at[k * W + j]], ssems[rb],
                                     add=True)

            def wait_s(k, rb):
                for j in range(W):
                    pltpu.make_async_copy(
                        rows.at[rb, pl.ds(j * 128, 128)],
                        acc.at[dstv.at[k * W + j]], ssems[rb]).wait()

            @pl.loop(0, ngrp)
            def _(gi):
                r0 = row0 + gi * GR
                pltpu.sync_copy(src_hbm.at[pl.ds(r0, GR)], sidx)
                pltpu.sync_copy(dst_hbm.at[pl.ds(r0, GR)], dstv)
                fire_g(0, 0)
                for k in range(GW):
                    rb = k % 2
                    if k >= 1:
                        wait_s(k - 1, rb ^ 1)
                    if k + 1 < GW:
                        fire_g(k + 1, rb ^ 1)
                    wait_g(k, rb)
                    fire_s(k, rb)
                wait_s(GW - 1, (GW - 1) % 2)

            plsc.subcore_barrier()
            for q in range(NCP):
                pltpu.sync_copy(acc.at[pl.ds(s * CL_PT + q * WE, WE)],
                                rows.at[1])
                pltpu.sync_copy(
                    rows.at[1], out_hbm.at[c, pl.ds(s * CL_PT + q * WE, WE)])
            pltpu.sync_copy(
                acc.at[pl.ds(s * CL_PT + NCP * WE, CL_TAIL)],
                rows.at[1, pl.ds(0, CL_TAIL)])
            pltpu.sync_copy(
                rows.at[1, pl.ds(0, CL_TAIL)],
                out_hbm.at[c, pl.ds(s * CL_PT + NCP * WE, CL_TAIL)])
            plsc.subcore_barrier()

    res = seg_kernel(*u_slices, src2d, dst2d)
    return list(res) if isinstance(res, (tuple, list)) else [res]


def _tc_first(x, W1, deg):
    BR = 2000

    def body(x_ref, w_ref, deg_ref, u_ref, dinv_ref):
        dg = deg_ref[0, :, :] + deg_ref[1, :, :]
        dinv = lax.rsqrt(dg + 1.0)
        xx = x_ref[...]
        w = w_ref[...]
        h = (xx[:, 0:1] * w[0:1, :] + xx[:, 1:2] * w[1:2, :]
             + xx[:, 2:3] * w[2:3, :])
        u_ref[...] = dinv * h
        dinv_ref[...] = dinv

    return pl.pallas_call(
        body,
        grid=(N // BR,),
        in_specs=[
            pl.BlockSpec((BR, 3), lambda i: (i, 0)),
            pl.BlockSpec((3, 16), lambda i: (0, 0)),
            pl.BlockSpec((2, BR, 1), lambda i: (0, i, 0)),
        ],
        out_specs=[
            pl.BlockSpec((BR, 16), lambda i: (i, 0)),
            pl.BlockSpec((BR, 1), lambda i: (i, 0)),
        ],
        out_shape=[
            jax.ShapeDtypeStruct((N, 16), _F32),
            jax.ShapeDtypeStruct((N, 1), _F32),
        ],
    )(x, W1, deg)


def _tc_stats(S_slices, u_slices, dinv, b2d, F):
    K = F // 16
    BR = 2000
    NB = N // BR

    def body(*refs):
        s_refs = refs[:K]
        u_refs = refs[K:2 * K]
        dinv_ref, b_ref, out_ref, acc = refs[2 * K:]
        i = pl.program_id(0)

        @pl.when(i == 0)
        def _():
            acc[...] = jnp.zeros_like(acc)

        dinv = dinv_ref[...]
        for sl in range(K):
            svals = s_refs[sl][0, :, :] + s_refs[sl][1, :, :]
            t = dinv * (svals + u_refs[sl][...]) + b_ref[:, sl * 16:sl * 16 + 16]
            acc[0:1, sl * 16:sl * 16 + 16] += jnp.sum(t, axis=0, keepdims=True)
            acc[1:2, sl * 16:sl * 16 + 16] += jnp.sum(t * t, axis=0,
                                                      keepdims=True)

        @pl.when(i == NB - 1)
        def _():
            out_ref[...] = acc[...]

    return pl.pallas_call(
        body,
        grid=(NB,),
        in_specs=(
            [pl.BlockSpec((2, BR, 16), lambda i: (0, i, 0)) for _ in range(K)]
            + [pl.BlockSpec((BR, 16), lambda i: (i, 0)) for _ in range(K)]
            + [pl.BlockSpec((BR, 1), lambda i: (i, 0)),
               pl.BlockSpec((1, F), lambda i: (0, 0))]
        ),
        out_specs=pl.BlockSpec((8, F), lambda i: (0, 0)),
        out_shape=jax.ShapeDtypeStruct((8, F), _F32),
        scratch_shapes=[pltpu.VMEM((8, F), _F32)],
    )(*S_slices, *u_slices, dinv, b2d)


def _tc_apply(S_slices, u_slices, dinv, b2d, g2d, be2d, stats, Wn, F, F2):
    K = F // 16
    K2 = F2 // 16
    BR = 2000

    def body(*refs):
        s_refs = refs[:K]
        u_refs = refs[K:2 * K]
        dinv_ref, b_ref, g_ref, be_ref, st_ref, w_ref = refs[2 * K:2 * K + 6]
        out_refs = refs[2 * K + 6:]
        dinv = dinv_ref[...]
        mu = st_ref[0:1, :] * (1.0 / N)
        var = st_ref[1:2, :] * (1.0 / N) - mu * mu
        a = g_ref[...] * lax.rsqrt(var + 1e-5)
        hs = []
        for sl in range(K):
            svals = s_refs[sl][0, :, :] + s_refs[sl][1, :, :]
            t = dinv * (svals + u_refs[sl][...]) + b_ref[:, sl * 16:sl * 16 + 16]
            hs.append(jnp.maximum(
                a[:, sl * 16:sl * 16 + 16] * (t - mu[:, sl * 16:sl * 16 + 16])
                + be_ref[:, sl * 16:sl * 16 + 16], 0.0))
        h = jnp.concatenate(hs, axis=1)
        un = dinv * jnp.dot(h, w_ref[...], preferred_element_type=_F32)
        for sl in range(K2):
            out_refs[sl][...] = un[:, sl * 16:sl * 16 + 16]

    return pl.pallas_call(
        body,
        grid=(N // BR,),
        in_specs=(
            [pl.BlockSpec((2, BR, 16), lambda i: (0, i, 0)) for _ in range(K)]
            + [pl.BlockSpec((BR, 16), lambda i: (i, 0)) for _ in range(K)]
            + [pl.BlockSpec((BR, 1), lambda i: (i, 0)),
               pl.BlockSpec((1, F), lambda i: (0, 0)),
               pl.BlockSpec((1, F), lambda i: (0, 0)),
               pl.BlockSpec((1, F), lambda i: (0, 0)),
               pl.BlockSpec((8, F), lambda i: (0, 0)),
               pl.BlockSpec((F, F2), lambda i: (0, 0))]
        ),
        out_specs=[pl.BlockSpec((BR, 16), lambda i: (i, 0))
                   for _ in range(K2)],
        out_shape=[jax.ShapeDtypeStruct((N, 16), _F32) for _ in range(K2)],
    )(*S_slices, *u_slices, dinv, b2d, g2d, be2d, stats, Wn)


def _tc_hist(batch2d):
    BR = 2000
    NB = N // BR

    def body(bt_ref, out_ref, acc):
        i = pl.program_id(0)

        @pl.when(i == 0)
        def _():
            acc[...] = jnp.zeros_like(acc)

        ids = bt_ref[...]
        gidx = lax.broadcasted_iota(jnp.int32, (BR, G), 1)
        acc[0:1, :] += jnp.sum((ids == gidx).astype(jnp.int32), axis=0,
                               keepdims=True)

        @pl.when(i == NB - 1)
        def _():
            out_ref[...] = acc[0:1, :]

    return pl.pallas_call(
        body,
        grid=(NB,),
        in_specs=[pl.BlockSpec((BR, 1), lambda i: (i, 0))],
        out_specs=pl.BlockSpec((1, G), lambda i: (0, 0)),
        out_shape=jax.ShapeDtypeStruct((1, G), jnp.int32),
        scratch_shapes=[pltpu.VMEM((8, G), jnp.int32)],
    )(batch2d)


def _tc_pool_head(S_slices, u_slices, dinv, b2d, g2d, be2d, stats, hist,
                  batch2d, Wf, bf2d):
    K = 4
    BR = 2000
    NB = N // BR
    F = 64
    CH = 64

    def body(*refs):
        s_refs = refs[:K]
        u_refs = refs[K:2 * K]
        (dinv_ref, b_ref, g_ref, be_ref, st_ref, ht_ref, bt_ref, wf_ref,
         bf_ref, lo_ref, pr_ref, hblk, starts, pacc) = refs[2 * K:]
        i = pl.program_id(0)

        @pl.when(i == 0)
        def _():
            sacc = 0
            for g in range(G):
                starts[g] = sacc
                sacc = sacc + ht_ref[0, g]
            starts[G] = sacc
            pacc[...] = jnp.full((G, F), -jnp.inf, _F32)

        dinv = dinv_ref[...]
        mu = st_ref[0:1, :] * (1.0 / N)
        var = st_ref[1:2, :] * (1.0 / N) - mu * mu
        a = g_ref[...] * lax.rsqrt(var + 1e-5)
        for sl in range(K):
            svals = s_refs[sl][0, :, :] + s_refs[sl][1, :, :]
            t = (dinv * (svals + u_refs[sl][...])
                 + b_ref[:, sl * 16:sl * 16 + 16])
            h = jnp.maximum(
                a[:, sl * 16:sl * 16 + 16]
                * (t - mu[:, sl * 16:sl * 16 + 16])
                + be_ref[:, sl * 16:sl * 16 + 16], 0.0)
            hblk[pl.ds(0, BR), sl * 16:sl * 16 + 16] = h

        blk_lo = i * BR
        gfirst = bt_ref[0, 0]
        glast = bt_ref[BR - 1, 0]

        def graph_body(g, carry):
            s_g = jnp.maximum(starts[g], blk_lo) - blk_lo
            e_g = jnp.minimum(starts[g + 1], blk_lo + BR) - blk_lo
            nch = (e_g - s_g + CH - 1) // CH

            def chunk_body(k, mx):
                j = s_g + k * CH
                hv = hblk[pl.ds(j, CH), :]
                rid = j + lax.broadcasted_iota(jnp.int32, (CH, F), 0)
                return jnp.maximum(mx, jnp.where(rid < e_g, hv, -jnp.inf))

            acc0 = jnp.full((CH, F), -jnp.inf, _F32)
            mx = lax.fori_loop(0, nch, chunk_body, acc0)
            cur = pacc[pl.ds(g, 1), :]
            pacc[pl.ds(g, 1), :] = jnp.maximum(
                cur, jnp.max(mx, axis=0, keepdims=True))
            return carry

        lax.fori_loop(gfirst, glast + 1, graph_body, 0)

        @pl.when(i == NB - 1)
        def _():
            p = pacc[...]
            logits = jnp.dot(p, wf_ref[...],
                             preferred_element_type=_F32) + bf_ref[...]
            lo_ref[...] = logits
            mxl = jnp.max(logits, axis=1, keepdims=True)
            e = jnp.exp(logits - mxl)
            pr_ref[...] = e / jnp.sum(e, axis=1, keepdims=True)

    return pl.pallas_call(
        body,
        grid=(NB,),
        in_specs=(
            [pl.BlockSpec((2, BR, 16), lambda i: (0, i, 0))
             for _ in range(K)]
            + [pl.BlockSpec((BR, 16), lambda i: (i, 0)) for _ in range(K)]
            + [pl.BlockSpec((BR, 1), lambda i: (i, 0)),
               pl.BlockSpec((1, F), lambda i: (0, 0)),
               pl.BlockSpec((1, F), lambda i: (0, 0)),
               pl.BlockSpec((1, F), lambda i: (0, 0)),
               pl.BlockSpec((8, F), lambda i: (0, 0)),
               pl.BlockSpec(memory_space=pltpu.SMEM),
               pl.BlockSpec((BR, 1), lambda i: (i, 0)),
               pl.BlockSpec((F, 10), lambda i: (0, 0)),
               pl.BlockSpec((1, 10), lambda i: (0, 0))]
        ),
        out_specs=[
            pl.BlockSpec((G, 10), lambda i: (0, 0)),
            pl.BlockSpec((G, 10), lambda i: (0, 0)),
        ],
        out_shape=[
            jax.ShapeDtypeStruct((G, 10), _F32),
            jax.ShapeDtypeStruct((G, 10), _F32),
        ],
        scratch_shapes=[
            pltpu.VMEM((BR + CH, F), _F32),
            pltpu.SMEM((G + 8,), jnp.int32),
            pltpu.VMEM((G, F), _F32),
        ],
    )(*S_slices, *u_slices, dinv, b2d, g2d, be2d, stats, hist, batch2d,
      Wf, bf2d)


def kernel(x, edge_index, batch, W1, b1, g1, be1, W2, b2, g2, be2,
           W3, b3, g3, be3, Wf, bf):
    src = edge_index[0].astype(jnp.int32)
    dst = edge_index[1].astype(jnp.int32)
    pad = EP - E
    pad_src = (jnp.arange(pad, dtype=jnp.int32) * 997) % N
    pad_dst = TRASH + (jnp.arange(pad, dtype=jnp.int32) % (NP - TRASH))
    src2d = jnp.concatenate([src, pad_src]).reshape(ER, 128)
    dst2d = jnp.concatenate([dst, pad_dst]).reshape(ER, 128)

    deg = _sc_degree(dst2d)
    deg3 = deg[:, :N, None]

    u1, dinv = _tc_first(x, W1, deg3)

    b1r, g1r, be1r = b1[None, :], g1[None, :], be1[None, :]
    b2r, g2r, be2r = b2[None, :], g2[None, :], be2[None, :]
    b3r, g3r, be3r = b3[None, :], g3[None, :], be3[None, :]
    bfr = bf[None, :]
    batch2d = batch.astype(jnp.int32)[:, None]

    S1 = _sc_segsum_slices([u1], src2d, dst2d)
    st1 = _tc_stats(S1, [u1], dinv, b1r, 16)
    u2s = _tc_apply(S1, [u1], dinv, b1r, g1r, be1r, st1, W2, 16, 32)

    S2 = _sc_segsum_slices(list(u2s), src2d, dst2d)
    st2 = _tc_stats(S2, list(u2s), dinv, b2r, 32)
    u3s = _tc_apply(S2, list(u2s), dinv, b2r, g2r, be2r, st2, W3, 32, 64)

    S3 = _sc_segsum_slices(list(u3s), src2d, dst2d)
    st3 = _tc_stats(S3, list(u3s), dinv, b3r, 64)
    hist = _tc_hist(batch2d)
    logits, probs = _tc_pool_head(S3, list(u3s), dinv, b3r, g3r, be3r, st3,
                                  hist, batch2d, Wf, bfr)
    return (logits, probs)


# speedup vs baseline: 1.5773x; 1.0753x over previous
import functools

import jax
import jax.numpy as jnp
from jax import lax
from jax.experimental import pallas as pl
from jax.experimental.pallas import tpu as pltpu
from jax.experimental.pallas import tpu_sc as plsc

N = 100000
E = 1600000
G = 128

NP = 100352
ER = 12800
EP = ER * 128
TRASH = 100000

_F32 = jnp.float32
_MESH = plsc.VectorSubcoreMesh(core_axis_name="c", subcore_axis_name="s")
_SC_PARAMS = pltpu.CompilerParams(use_tc_tiling_on_sc=False)


def _sc_degree(dst2d):
    CL_PT = NP // 16
    ZD = 784
    NCPD = CL_PT // ZD
    TILE_ROWS = ER // 32
    W_ROWS = 16
    N_IT = TILE_ROWS // W_ROWS

    @functools.partial(
        pl.kernel,
        out_type=jax.ShapeDtypeStruct((2, NP), _F32),
        mesh=_MESH,
        compiler_params=_SC_PARAMS,
        scratch_types=[
            pltpu.VMEM((W_ROWS, 128), jnp.int32),
            pltpu.VMEM((128,), _F32),
            pltpu.VMEM((ZD,), _F32),
            pltpu.VMEM_SHARED((NP,), _F32),
            pltpu.SemaphoreType.DMA,
        ],
    )
    def deg_kernel(dst_hbm, out_hbm, dstv, ones, stg, acc, ssem):
        c = lax.axis_index("c")
        s = lax.axis_index("s")

        @pl.loop(0, 128, step=16)
        def _(k):
            ones[pl.ds(k, 16)] = jnp.full((16,), 1.0, _F32)

        @pl.loop(0, ZD, step=16)
        def _(k):
            stg[pl.ds(k, 16)] = jnp.zeros((16,), _F32)

        for q in range(NCPD):
            pltpu.sync_copy(stg, acc.at[pl.ds(s * CL_PT + q * ZD, ZD)])
        plsc.subcore_barrier()

        @pl.loop(0, N_IT)
        def _(it):
            r0 = c * (ER // 2) + s * TILE_ROWS + it * W_ROWS
            pltpu.sync_copy(dst_hbm.at[pl.ds(r0, W_ROWS)], dstv)
            copies = [
                pltpu.async_copy(ones, acc.at[dstv.at[j]], ssem, add=True)
                for j in range(W_ROWS)
            ]
            for cp in copies:
                cp.wait()

        plsc.subcore_barrier()
        for q in range(NCPD):
            pltpu.sync_copy(acc.at[pl.ds(s * CL_PT + q * ZD, ZD)], stg)
            pltpu.sync_copy(
                stg, out_hbm.at[c, pl.ds(s * CL_PT + q * ZD, ZD)])

    return deg_kernel(dst2d)


def _sc_segsum_slices(u_slices, src2d, dst2d):
    K = len(u_slices)
    SF = 16
    W = 4
    WE = W * 128
    GW = 10
    GR = GW * W
    R0 = ER // 32
    R1 = ER // 16 - R0
    CL_PT = NP // 16
    NCP = CL_PT // WE
    CL_TAIL = CL_PT - NCP * WE

    @functools.partial(
        pl.kernel,
        out_type=[jax.ShapeDtypeStruct((2, NP, SF), _F32) for _ in range(K)],
        mesh=_MESH,
        compiler_params=_SC_PARAMS,
        scratch_types=[
            pltpu.VMEM((GR, 128), jnp.int32),
            pltpu.VMEM((GR, 128), jnp.int32),
            pltpu.VMEM((2, WE, SF), _F32),
            pltpu.VMEM_SHARED((NP, SF), _F32),
            pltpu.SemaphoreType.DMA,
            pltpu.SemaphoreType.DMA,
            pltpu.SemaphoreType.DMA,
            pltpu.SemaphoreType.DMA,
        ],
    )
    def seg_kernel(*refs):
        us = refs[:K]
        src_hbm, dst_hbm = refs[K], refs[K + 1]
        outs = refs[K + 2:2 * K + 2]
        sidx, dstv, rows, acc, g0, g1, s0, s1 = refs[2 * K + 2:]
        gsems = (g0, g1)
        ssems = (s0, s1)
        c = lax.axis_index("c")
        s = lax.axis_index("s")
        row0 = jnp.where(c == 0, s * R0, 16 * R0 + s * R1)
        ngrp = jnp.where(c == 0, R0 // GR, R1 // GR)

        for sl in range(K):
            u_hbm = us[sl]
            out_hbm = outs[sl]

            @pl.loop(0, WE)
            def _(r):
                rows[0, r, pl.ds(0, 16)] = jnp.zeros((16,), _F32)

            for q in range(NCP):
                pltpu.sync_copy(rows.at[0],
                                acc.at[pl.ds(s * CL_PT + q * WE, WE)])
            pltpu.sync_copy(
                rows.at[0, pl.ds(0, CL_TAIL)],
                acc.at[pl.ds(s * CL_PT + NCP * WE, CL_TAIL)])
            plsc.subcore_barrier()

            def fire_g(k, rb, u_hbm=u_hbm):
                for j in range(W):
                    pltpu.async_copy(u_hbm.at[sidx.at[k * W + j]],
                                     rows.at[rb, pl.ds(j * 128, 128)],
                                     gsems[rb])

            def wait_g(k, rb, u_hbm=u_hbm):
                for j in range(W):
                    pltpu.make_async_copy(
                        u_hbm.at[sidx.at[k * W + j]],
                        rows.at[rb, pl.ds(j * 128, 128)], gsems[rb]).wait()

            def fire_s(k, rb):
                for j in range(W):
                    pltpu.async_copy(rows.at[rb, pl.ds(j * 128, 128)],
                                     acc.at[dstv.at[k * W + j]], ssems[rb],
                                     add=True)

            def wait_s(k, rb):
                for j in range(W):
                    pltpu.make_async_copy(
                        rows.at[rb, pl.ds(j * 128, 128)],
                        acc.at[dstv.at[k * W + j]], ssems[rb]).wait()

            @pl.loop(0, ngrp)
            def _(gi):
                r0 = row0 + gi * GR
                pltpu.sync_copy(src_hbm.at[pl.ds(r0, GR)], sidx)
                pltpu.sync_copy(dst_hbm.at[pl.ds(r0, GR)], dstv)
                fire_g(0, 0)
                for k in range(GW):
                    rb = k % 2
                    if k >= 1:
                        wait_s(k - 1, rb ^ 1)
                    if k + 1 < GW:
                        fire_g(k + 1, rb ^ 1)
                    wait_g(k, rb)
                    fire_s(k, rb)
                wait_s(GW - 1, (GW - 1) % 2)

            plsc.subcore_barrier()
            for q in range(NCP):
                pltpu.sync_copy(acc.at[pl.ds(s * CL_PT + q * WE, WE)],
                                rows.at[1])
                pltpu.sync_copy(
                    rows.at[1], out_hbm.at[c, pl.ds(s * CL_PT + q * WE, WE)])
            pltpu.sync_copy(
                acc.at[pl.ds(s * CL_PT + NCP * WE, CL_TAIL)],
                rows.at[1, pl.ds(0, CL_TAIL)])
            pltpu.sync_copy(
                rows.at[1, pl.ds(0, CL_TAIL)],
                out_hbm.at[c, pl.ds(s * CL_PT + NCP * WE, CL_TAIL)])
            plsc.subcore_barrier()

    res = seg_kernel(*u_slices, src2d, dst2d)
    return list(res) if isinstance(res, (tuple, list)) else [res]


def _tc_first(x, W1, deg):
    BR = 2000

    def body(x_ref, w_ref, deg_ref, u_ref, dinv_ref):
        dg = deg_ref[0, :, :] + deg_ref[1, :, :]
        dinv = lax.rsqrt(dg + 1.0)
        xx = x_ref[...]
        w = w_ref[...]
        h = (xx[:, 0:1] * w[0:1, :] + xx[:, 1:2] * w[1:2, :]
             + xx[:, 2:3] * w[2:3, :])
        u_ref[...] = dinv * h
        dinv_ref[...] = dinv

    return pl.pallas_call(
        body,
        grid=(N // BR,),
        in_specs=[
            pl.BlockSpec((BR, 3), lambda i: (i, 0)),
            pl.BlockSpec((3, 16), lambda i: (0, 0)),
            pl.BlockSpec((2, BR, 1), lambda i: (0, i, 0)),
        ],
        out_specs=[
            pl.BlockSpec((BR, 16), lambda i: (i, 0)),
            pl.BlockSpec((BR, 1), lambda i: (i, 0)),
        ],
        out_shape=[
            jax.ShapeDtypeStruct((N, 16), _F32),
            jax.ShapeDtypeStruct((N, 1), _F32),
        ],
    )(x, W1, deg)


def _tc_stats(S_slices, u_slices, dinv, b2d, F):
    K = F // 16
    BR = 2000
    NB = N // BR

    def body(*refs):
        s_refs = refs[:K]
        u_refs = refs[K:2 * K]
        dinv_ref, b_ref, out_ref, acc = refs[2 * K:]
        i = pl.program_id(0)

        @pl.when(i == 0)
        def _():
            acc[...] = jnp.zeros_like(acc)

        dinv = dinv_ref[...]
        for sl in range(K):
            svals = s_refs[sl][0, :, :] + s_refs[sl][1, :, :]
            t = dinv * (svals + u_refs[sl][...]) + b_ref[:, sl * 16:sl * 16 + 16]
            acc[0:1, sl * 16:sl * 16 + 16] += jnp.sum(t, axis=0, keepdims=True)
            acc[1:2, sl * 16:sl * 16 + 16] += jnp.sum(t * t, axis=0,
                                                      keepdims=True)

        @pl.when(i == NB - 1)
        def _():
            out_ref[...] = acc[...]

    return pl.pallas_call(
        body,
        grid=(NB,),
        in_specs=(
            [pl.BlockSpec((2, BR, 16), lambda i: (0, i, 0)) for _ in range(K)]
            + [pl.BlockSpec((BR, 16), lambda i: (i, 0)) for _ in range(K)]
            + [pl.BlockSpec((BR, 1), lambda i: (i, 0)),
               pl.BlockSpec((1, F), lambda i: (0, 0))]
        ),
        out_specs=pl.BlockSpec((8, F), lambda i: (0, 0)),
        out_shape=jax.ShapeDtypeStruct((8, F), _F32),
        scratch_shapes=[pltpu.VMEM((8, F), _F32)],
    )(*S_slices, *u_slices, dinv, b2d)


def _tc_apply(S_slices, u_slices, dinv, b2d, g2d, be2d, stats, Wn, F, F2):
    K = F // 16
    K2 = F2 // 16
    BR = 2000

    def body(*refs):
        s_refs = refs[:K]
        u_refs = refs[K:2 * K]
        dinv_ref, b_ref, g_ref, be_ref, st_ref, w_ref = refs[2 * K:2 * K + 6]
        out_refs = refs[2 * K + 6:]
        dinv = dinv_ref[...]
        mu = st_ref[0:1, :] * (1.0 / N)
        var = st_ref[1:2, :] * (1.0 / N) - mu * mu
        a = g_ref[...] * lax.rsqrt(var + 1e-5)
        hs = []
        for sl in range(K):
            svals = s_refs[sl][0, :, :] + s_refs[sl][1, :, :]
            t = dinv * (svals + u_refs[sl][...]) + b_ref[:, sl * 16:sl * 16 + 16]
            hs.append(jnp.maximum(
                a[:, sl * 16:sl * 16 + 16] * (t - mu[:, sl * 16:sl * 16 + 16])
                + be_ref[:, sl * 16:sl * 16 + 16], 0.0))
        h = jnp.concatenate(hs, axis=1)
        un = dinv * jnp.dot(h, w_ref[...], preferred_element_type=_F32)
        for sl in range(K2):
            out_refs[sl][...] = un[:, sl * 16:sl * 16 + 16]

    return pl.pallas_call(
        body,
        grid=(N // BR,),
        in_specs=(
            [pl.BlockSpec((2, BR, 16), lambda i: (0, i, 0)) for _ in range(K)]
            + [pl.BlockSpec((BR, 16), lambda i: (i, 0)) for _ in range(K)]
            + [pl.BlockSpec((BR, 1), lambda i: (i, 0)),
               pl.BlockSpec((1, F), lambda i: (0, 0)),
               pl.BlockSpec((1, F), lambda i: (0, 0)),
               pl.BlockSpec((1, F), lambda i: (0, 0)),
               pl.BlockSpec((8, F), lambda i: (0, 0)),
               pl.BlockSpec((F, F2), lambda i: (0, 0))]
        ),
        out_specs=[pl.BlockSpec((BR, 16), lambda i: (i, 0))
                   for _ in range(K2)],
        out_shape=[jax.ShapeDtypeStruct((N, 16), _F32) for _ in range(K2)],
    )(*S_slices, *u_slices, dinv, b2d, g2d, be2d, stats, Wn)


def _tc_hist(batch2d):
    BR = 2000
    NB = N // BR

    def body(bt_ref, out_ref, acc):
        i = pl.program_id(0)

        @pl.when(i == 0)
        def _():
            acc[...] = jnp.zeros_like(acc)

        ids = bt_ref[...]
        gidx = lax.broadcasted_iota(jnp.int32, (BR, G), 1)
        acc[0:1, :] += jnp.sum((ids == gidx).astype(jnp.int32), axis=0,
                               keepdims=True)

        @pl.when(i == NB - 1)
        def _():
            out_ref[...] = acc[0:1, :]

    return pl.pallas_call(
        body,
        grid=(NB,),
        in_specs=[pl.BlockSpec((BR, 1), lambda i: (i, 0))],
        out_specs=pl.BlockSpec((1, G), lambda i: (0, 0)),
        out_shape=jax.ShapeDtypeStruct((1, G), jnp.int32),
        scratch_shapes=[pltpu.VMEM((8, G), jnp.int32)],
    )(batch2d)


def _tc_pool_head(S_slices, u_slices, dinv, b2d, g2d, be2d, stats, hist,
                  batch2d, Wf, bf2d):
    K = 4
    BR = 2000
    NB = N // BR
    F = 64
    CH = 64

    def body(*refs):
        s_refs = refs[:K]
        u_refs = refs[K:2 * K]
        (dinv_ref, b_ref, g_ref, be_ref, st_ref, ht_ref, bt_ref, wf_ref,
         bf_ref, lo_ref, pr_ref, hblk, starts, pacc) = refs[2 * K:]
        i = pl.program_id(0)

        @pl.when(i == 0)
        def _():
            sacc = 0
            for g in range(G):
                starts[g] = sacc
                sacc = sacc + ht_ref[0, g]
            starts[G] = sacc
            pacc[...] = jnp.full((G, F), -jnp.inf, _F32)

        dinv = dinv_ref[...]
        mu = st_ref[0:1, :] * (1.0 / N)
        var = st_ref[1:2, :] * (1.0 / N) - mu * mu
        a = g_ref[...] * lax.rsqrt(var + 1e-5)
        for sl in range(K):
            svals = s_refs[sl][0, :, :] + s_refs[sl][1, :, :]
            t = (dinv * (svals + u_refs[sl][...])
                 + b_ref[:, sl * 16:sl * 16 + 16])
            h = jnp.maximum(
                a[:, sl * 16:sl * 16 + 16]
                * (t - mu[:, sl * 16:sl * 16 + 16])
                + be_ref[:, sl * 16:sl * 16 + 16], 0.0)
            hblk[pl.ds(0, BR), sl * 16:sl * 16 + 16] = h

        blk_lo = i * BR
        gfirst = bt_ref[0, 0]
        glast = bt_ref[BR - 1, 0]

        def graph_body(g, carry):
            s_g = jnp.maximum(starts[g], blk_lo) - blk_lo
            e_g = jnp.minimum(starts[g + 1], blk_lo + BR) - blk_lo
            nch = (e_g - s_g + CH - 1) // CH

            def chunk_body(k, mx):
                j = s_g + k * CH
                hv = hblk[pl.ds(j, CH), :]
                rid = j + lax.broadcasted_iota(jnp.int32, (CH, F), 0)
                return jnp.maximum(mx, jnp.where(rid < e_g, hv, -jnp.inf))

            acc0 = jnp.full((CH, F), -jnp.inf, _F32)
            mx = lax.fori_loop(0, nch, chunk_body, acc0)
            cur = pacc[pl.ds(g, 1), :]
            pacc[pl.ds(g, 1), :] = jnp.maximum(
                cur, jnp.max(mx, axis=0, keepdims=True))
            return carry

        lax.fori_loop(gfirst, glast + 1, graph_body, 0)

        @pl.when(i == NB - 1)
        def _():
            p = pacc[...]
            logits = jnp.dot(p, wf_ref[...],
                             preferred_element_type=_F32) + bf_ref[...]
            lo_ref[...] = logits
            mxl = jnp.max(logits, axis=1, keepdims=True)
            e = jnp.exp(logits - mxl)
            pr_ref[...] = e / jnp.sum(e, axis=1, keepdims=True)

    return pl.pallas_call(
        body,
        grid=(NB,),
        in_specs=(
            [pl.BlockSpec((2, BR, 16), lambda i: (0, i, 0))
             for _ in range(K)]
            + [pl.BlockSpec((BR, 16), lambda i: (i, 0)) for _ in range(K)]
            + [pl.BlockSpec((BR, 1), lambda i: (i, 0)),
               pl.BlockSpec((1, F), lambda i: (0, 0)),
               pl.BlockSpec((1, F), lambda i: (0, 0)),
               pl.BlockSpec((1, F), lambda i: (0, 0)),
               pl.BlockSpec((8, F), lambda i: (0, 0)),
               pl.BlockSpec(memory_space=pltpu.SMEM),
               pl.BlockSpec((BR, 1), lambda i: (i, 0)),
               pl.BlockSpec((F, 10), lambda i: (0, 0)),
               pl.BlockSpec((1, 10), lambda i: (0, 0))]
        ),
        out_specs=[
            pl.BlockSpec((G, 10), lambda i: (0, 0)),
            pl.BlockSpec((G, 10), lambda i: (0, 0)),
        ],
        out_shape=[
            jax.ShapeDtypeStruct((G, 10), _F32),
            jax.ShapeDtypeStruct((G, 10), _F32),
        ],
        scratch_shapes=[
            pltpu.VMEM((BR + CH, F), _F32),
            pltpu.SMEM((G + 8,), jnp.int32),
            pltpu.VMEM((G, F), _F32),
        ],
    )(*S_slices, *u_slices, dinv, b2d, g2d, be2d, stats, hist, batch2d,
      Wf, bf2d)


def kernel(x, edge_index, batch, W1, b1, g1, be1, W2, b2, g2, be2,
           W3, b3, g3, be3, Wf, bf):
    src = edge_index[0].astype(jnp.int32)
    dst = edge_index[1].astype(jnp.int32)
    pad = EP - E
    pad_src = (jnp.arange(pad, dtype=jnp.int32) * 997) % N
    pad_dst = TRASH + (jnp.arange(pad, dtype=jnp.int32) % (NP - TRASH))
    src2d = jnp.concatenate([src, pad_src]).reshape(ER, 128)
    dst2d = jnp.concatenate([dst, pad_dst]).reshape(ER, 128)

    deg = _sc_degree(dst2d)
    deg3 = deg[:, :N, None]

    u1, dinv = _tc_first(x, W1, deg3)

    b1r, g1r, be1r = b1[None, :], g1[None, :], be1[None, :]
    b2r, g2r, be2r = b2[None, :], g2[None, :], be2[None, :]
    b3r, g3r, be3r = b3[None, :], g3[None, :], be3[None, :]
    bfr = bf[None, :]
    batch2d = batch.astype(jnp.int32)[:, None]

    S1 = _sc_segsum_slices([u1], src2d, dst2d)
    st1 = _tc_stats(S1, [u1], dinv, b1r, 16)
    u2s = _tc_apply(S1, [u1], dinv, b1r, g1r, be1r, st1, W2, 16, 32)

    S2 = _sc_segsum_slices(list(u2s), src2d, dst2d)
    st2 = _tc_stats(S2, list(u2s), dinv, b2r, 32)
    u3s = _tc_apply(S2, list(u2s), dinv, b2r, g2r, be2r, st2, W3, 32, 64)

    S3 = _sc_segsum_slices(list(u3s), src2d, dst2d)
    st3 = _tc_stats(S3, list(u3s), dinv, b3r, 64)
    hist = _tc_hist(batch2d)
    logits, probs = _tc_pool_head(S3, list(u3s), dinv, b3r, g3r, be3r, st3,
                                  hist, batch2d, Wf, bfr)
    return (logits, probs)
